# baseline (speedup 1.0000x reference)
import jax
import jax.numpy as jnp
from jax.experimental import pallas as pl
from jax.experimental.pallas import tpu as pltpu

N_CHUNKS = 4
USER_CHUNK = 25000
ITEM_CHUNK = 12500


def _copy_kernel(user_in, item_in, user_out, item_out, sems):
    copies = []
    for c in range(N_CHUNKS):
        copies.append(pltpu.make_async_copy(
            user_in.at[pl.ds(c * USER_CHUNK, USER_CHUNK), :],
            user_out.at[pl.ds(c * USER_CHUNK, USER_CHUNK), :],
            sems.at[c]))
        copies.append(pltpu.make_async_copy(
            item_in.at[pl.ds(c * ITEM_CHUNK, ITEM_CHUNK), :],
            item_out.at[pl.ds(c * ITEM_CHUNK, ITEM_CHUNK), :],
            sems.at[N_CHUNKS + c]))
    for cp in copies:
        cp.start()
    for cp in copies:
        cp.wait()


def kernel(embed_user, embed_item):
    return tuple(pl.pallas_call(
        _copy_kernel,
        in_specs=[
            pl.BlockSpec(memory_space=pl.ANY),
            pl.BlockSpec(memory_space=pl.ANY),
        ],
        out_specs=[
            pl.BlockSpec(memory_space=pl.ANY),
            pl.BlockSpec(memory_space=pl.ANY),
        ],
        out_shape=[
            jax.ShapeDtypeStruct(embed_user.shape, embed_user.dtype),
            jax.ShapeDtypeStruct(embed_item.shape, embed_item.dtype),
        ],
        scratch_shapes=[pltpu.SemaphoreType.DMA((2 * N_CHUNKS,))],
    )(embed_user, embed_item))


# speedup vs baseline: 45.5281x; 45.5281x over previous
import jax
import jax.numpy as jnp
from jax.experimental import pallas as pl

N_GRID = 25
USER_ROWS = 4000
ITEM_ROWS = 2000
EMBED = 128


def _copy_kernel(user_in, item_in, user_out, item_out):
    user_out[...] = user_in[...]
    item_out[...] = item_in[...]


def kernel(embed_user, embed_item):
    return tuple(pl.pallas_call(
        _copy_kernel,
        grid=(N_GRID,),
        in_specs=[
            pl.BlockSpec((USER_ROWS, EMBED), lambda i: (i, 0)),
            pl.BlockSpec((ITEM_ROWS, EMBED), lambda i: (i, 0)),
        ],
        out_specs=[
            pl.BlockSpec((USER_ROWS, EMBED), lambda i: (i, 0)),
            pl.BlockSpec((ITEM_ROWS, EMBED), lambda i: (i, 0)),
        ],
        out_shape=[
            jax.ShapeDtypeStruct(embed_user.shape, embed_user.dtype),
            jax.ShapeDtypeStruct(embed_item.shape, embed_item.dtype),
        ],
    )(embed_user, embed_item))


# speedup vs baseline: 48.3014x; 1.0609x over previous
import jax
import jax.numpy as jnp
from jax.experimental import pallas as pl

N_GRID = 10
USER_ROWS = 10000
ITEM_ROWS = 5000
EMBED = 128


def _copy_kernel(user_in, item_in, user_out, item_out):
    user_out[...] = user_in[...]
    item_out[...] = item_in[...]


def kernel(embed_user, embed_item):
    return tuple(pl.pallas_call(
        _copy_kernel,
        grid=(N_GRID,),
        in_specs=[
            pl.BlockSpec((USER_ROWS, EMBED), lambda i: (i, 0)),
            pl.BlockSpec((ITEM_ROWS, EMBED), lambda i: (i, 0)),
        ],
        out_specs=[
            pl.BlockSpec((USER_ROWS, EMBED), lambda i: (i, 0)),
            pl.BlockSpec((ITEM_ROWS, EMBED), lambda i: (i, 0)),
        ],
        out_shape=[
            jax.ShapeDtypeStruct(embed_user.shape, embed_user.dtype),
            jax.ShapeDtypeStruct(embed_item.shape, embed_item.dtype),
        ],
    )(embed_user, embed_item))


# speedup vs baseline: 48.4381x; 1.0028x over previous
import jax
import jax.numpy as jnp
from jax.experimental import pallas as pl

N_GRID = 8
USER_ROWS = 12504
ITEM_ROWS = 6256
EMBED = 128


def _copy_kernel(user_in, item_in, user_out, item_out):
    user_out[...] = user_in[...]
    item_out[...] = item_in[...]


def kernel(embed_user, embed_item):
    return tuple(pl.pallas_call(
        _copy_kernel,
        grid=(N_GRID,),
        in_specs=[
            pl.BlockSpec((USER_ROWS, EMBED), lambda i: (i, 0)),
            pl.BlockSpec((ITEM_ROWS, EMBED), lambda i: (i, 0)),
        ],
        out_specs=[
            pl.BlockSpec((USER_ROWS, EMBED), lambda i: (i, 0)),
            pl.BlockSpec((ITEM_ROWS, EMBED), lambda i: (i, 0)),
        ],
        out_shape=[
            jax.ShapeDtypeStruct(embed_user.shape, embed_user.dtype),
            jax.ShapeDtypeStruct(embed_item.shape, embed_item.dtype),
        ],
    )(embed_user, embed_item))


# speedup vs baseline: 48.9683x; 1.0109x over previous
import jax
import jax.numpy as jnp
from jax.experimental import pallas as pl

N_GRID = 6
USER_ROWS = 16672
ITEM_ROWS = 8336
EMBED = 128


def _copy_kernel(user_in, item_in, user_out, item_out):
    user_out[...] = user_in[...]
    item_out[...] = item_in[...]


def kernel(embed_user, embed_item):
    return tuple(pl.pallas_call(
        _copy_kernel,
        grid=(N_GRID,),
        in_specs=[
            pl.BlockSpec((USER_ROWS, EMBED), lambda i: (i, 0)),
            pl.BlockSpec((ITEM_ROWS, EMBED), lambda i: (i, 0)),
        ],
        out_specs=[
            pl.BlockSpec((USER_ROWS, EMBED), lambda i: (i, 0)),
            pl.BlockSpec((ITEM_ROWS, EMBED), lambda i: (i, 0)),
        ],
        out_shape=[
            jax.ShapeDtypeStruct(embed_user.shape, embed_user.dtype),
            jax.ShapeDtypeStruct(embed_item.shape, embed_item.dtype),
        ],
    )(embed_user, embed_item))


# speedup vs baseline: 49.1103x; 1.0029x over previous
import jax
import jax.numpy as jnp
from jax.experimental import pallas as pl

N_GRID = 5
USER_ROWS = 20000
ITEM_ROWS = 10000
EMBED = 128


def _copy_kernel(user_in, item_in, user_out, item_out):
    user_out[...] = user_in[...]
    item_out[...] = item_in[...]


def kernel(embed_user, embed_item):
    return tuple(pl.pallas_call(
        _copy_kernel,
        grid=(N_GRID,),
        in_specs=[
            pl.BlockSpec((USER_ROWS, EMBED), lambda i: (i, 0)),
            pl.BlockSpec((ITEM_ROWS, EMBED), lambda i: (i, 0)),
        ],
        out_specs=[
            pl.BlockSpec((USER_ROWS, EMBED), lambda i: (i, 0)),
            pl.BlockSpec((ITEM_ROWS, EMBED), lambda i: (i, 0)),
        ],
        out_shape=[
            jax.ShapeDtypeStruct(embed_user.shape, embed_user.dtype),
            jax.ShapeDtypeStruct(embed_item.shape, embed_item.dtype),
        ],
    )(embed_user, embed_item))
